# Initial kernel scaffold; baseline (speedup 1.0000x reference)
import jax
import jax.numpy as jnp
from jax.experimental import pallas as pl


def kernel(x, gW1, gb1, gW2, gb2, eW1, eb1, eW2, eb2, train):
    raise NotImplementedError("write your pallas kernel here")


# speedup vs baseline: 2.0644x; 2.0644x over previous
import functools

import jax
import jax.numpy as jnp
from jax.experimental import pallas as pl
from jax.experimental.pallas import tpu as pltpu

N, D, H, E, K = 2048, 1024, 1024, 8, 2


def _moe_kernel(x_ref, gW1_ref, gb1_ref, gW2_ref, gb2_ref,
                eW1_ref, eb1_ref, eW2_ref, eb2_ref,
                out_ref, gates_ref):
    e = pl.program_id(0)

    @pl.when(e == 0)
    def _gating():
        xg = x_ref[...]
        l1 = jnp.tanh(
            jax.lax.dot_general(xg, gW1_ref[...], (((1,), (0,)), ((), ())),
                                preferred_element_type=jnp.float32)
            + gb1_ref[...][None, :])
        logits = jax.lax.dot_general(l1, gW2_ref[...], (((1,), (0,)), ((), ())),
                                     preferred_element_type=jnp.float32) \
            + gb2_ref[...][None, :]
        m = jnp.max(logits, axis=-1, keepdims=True)
        ex = jnp.exp(logits - m)
        probs = ex / jnp.sum(ex, axis=-1, keepdims=True)
        iota = jax.lax.broadcasted_iota(jnp.int32, probs.shape, 1)
        i1 = jnp.argmax(probs, axis=-1, keepdims=True)
        v1 = jnp.max(probs, axis=-1, keepdims=True)
        probs2 = jnp.where(iota == i1, -jnp.inf, probs)
        i2 = jnp.argmax(probs2, axis=-1, keepdims=True)
        v2 = jnp.max(probs2, axis=-1, keepdims=True)
        gates = jnp.where(iota == i1, v1, 0.0) + jnp.where(iota == i2, v2, 0.0)
        gates_ref[...] = gates

    xv = x_ref[...]
    w1 = eW1_ref[0]
    h = jnp.tanh(
        jax.lax.dot_general(xv, w1, (((1,), (0,)), ((), ())),
                            preferred_element_type=jnp.float32)
        + eb1_ref[0])
    y = jax.lax.dot_general(h, eW2_ref[0], (((1,), (0,)), ((), ())),
                            preferred_element_type=jnp.float32) \
        + eb2_ref[0]
    gates = gates_ref[...]
    lane = jax.lax.broadcasted_iota(jnp.int32, gates.shape, 1)
    g = jnp.sum(jnp.where(lane == e, gates, 0.0), axis=1, keepdims=True)
    contrib = y * g

    @pl.when(e == 0)
    def _init():
        out_ref[...] = contrib

    @pl.when(e != 0)
    def _acc():
        out_ref[...] += contrib


@jax.jit
def _moe(x, gW1, gb1, gW2, gb2, eW1, eb1, eW2, eb2):
    return pl.pallas_call(
        _moe_kernel,
        grid=(E,),
        in_specs=[
            pl.BlockSpec((N, D), lambda e: (0, 0)),
            pl.BlockSpec((D, E), lambda e: (0, 0)),
            pl.BlockSpec((E,), lambda e: (0,)),
            pl.BlockSpec((E, E), lambda e: (0, 0)),
            pl.BlockSpec((E,), lambda e: (0,)),
            pl.BlockSpec((1, D, H), lambda e: (e, 0, 0)),
            pl.BlockSpec((1, 1, H), lambda e: (e, 0, 0)),
            pl.BlockSpec((1, H, D), lambda e: (e, 0, 0)),
            pl.BlockSpec((1, 1, D), lambda e: (e, 0, 0)),
        ],
        out_specs=pl.BlockSpec((N, D), lambda e: (0, 0)),
        out_shape=jax.ShapeDtypeStruct((N, D), jnp.float32),
        scratch_shapes=[pltpu.VMEM((N, E), jnp.float32)],
        compiler_params=pltpu.CompilerParams(
            dimension_semantics=("arbitrary",),
        ),
    )(x, gW1, gb1, gW2, gb2, eW1, eb1[:, None, :], eW2, eb2[:, None, :])


def kernel(x, gW1, gb1, gW2, gb2, eW1, eb1, eW2, eb2, train):
    del train
    return _moe(x, gW1, gb1, gW2, gb2, eW1, eb1, eW2, eb2)
